# baseline (speedup 1.0000x reference)
import functools

import jax
import jax.numpy as jnp
from jax import lax
from jax.experimental import pallas as pl
from jax.experimental.pallas import tpu as pltpu
from jax.experimental.pallas import tpu_sc as plsc

NUM_SPECIES = 100
EMBED_DIM = 16
N_NODES = 100000

NC = 2
NS = 16
NW = NC * NS

B_MAIN = 3200
B_TAIL = N_NODES - 31 * B_MAIN


@functools.cache
def _make_gather():
    mesh = plsc.VectorSubcoreMesh(
        core_axis_name="c", subcore_axis_name="s", num_cores=NC, num_subcores=NS
    )

    @functools.partial(
        pl.kernel,
        out_type=jax.ShapeDtypeStruct((EMBED_DIM, N_NODES), jnp.float32),
        mesh=mesh,
        scratch_types=[
            pltpu.VMEM((EMBED_DIM, NUM_SPECIES), jnp.float32),
            pltpu.VMEM((NUM_SPECIES * EMBED_DIM + 16,), jnp.float32),
            pltpu.VMEM((B_MAIN,), jnp.int32),
            pltpu.VMEM((EMBED_DIM, B_MAIN), jnp.float32),
            pltpu.SemaphoreType.DMA,
        ],
        compiler_params=pltpu.CompilerParams(
            use_tc_tiling_on_sc=True, needs_layout_passes=False
        ),
    )
    def gather_kernel(
        table_hbm, idx_hbm, out_hbm, table2d_v, table_v, idx_v, col_v, sem
    ):
        wid = lax.axis_index("s") * NC + lax.axis_index("c")
        base = wid * B_MAIN
        pltpu.sync_copy(table_hbm, table2d_v)

        lanes16 = lax.iota(jnp.int32, 16)
        for c in range(EMBED_DIM):
            cvec = jnp.full((16,), c, jnp.int32)
            for k in range(7):
                s_idx = jnp.minimum(k * 16 + lanes16, NUM_SPECIES - 1)
                v = plsc.load_gather(table2d_v, [cvec, s_idx])
                table_v[pl.ds(c * NUM_SPECIES + k * 16, 16)] = v

        def load_group(g16):
            idx16 = idx_v[pl.ds(g16, 16)]
            return tuple(
                plsc.load_gather(table_v, [idx16 + (c * NUM_SPECIES)])
                for c in range(EMBED_DIM)
            )

        def store_group(g16, vs):
            for c in range(EMBED_DIM):
                col_v[c, pl.ds(g16, 16)] = vs[c]

        def do_rows(row0, count):
            def group_body(g, carry):
                prev_g16, prev = carry
                g16 = row0 + g * 16
                new = load_group(g16)
                store_group(prev_g16, prev)
                return (g16, new)

            carry = (row0, load_group(row0))
            carry = lax.fori_loop(1, count // 16, group_body, carry)
            store_group(*carry)

        def do_chunk(parts):
            n_rows = sum(parts)
            pltpu.sync_copy(
                idx_hbm.at[pl.ds(base, n_rows)], idx_v.at[pl.ds(0, n_rows)]
            )
            copies = []
            row0 = 0
            for part in parts:
                do_rows(row0, part)
                n_dma = ((part + 127) // 128) * 128
                copies.append(
                    pltpu.async_copy(
                        col_v.at[:, pl.ds(row0, n_dma)],
                        out_hbm.at[:, pl.ds(base + row0, n_dma)],
                        sem,
                    )
                )
                row0 += part
            for cp in copies:
                cp.wait()

        @pl.when(wid < NW - 1)
        def _():
            do_chunk((896, 768, 768, 768))

        @pl.when(wid == NW - 1)
        def _():
            do_chunk((B_TAIL,))

    return gather_kernel


@jax.jit
def kernel(species_index, embedding_table):
    out_t = _make_gather()(embedding_table.T, species_index.astype(jnp.int32))
    return out_t.T


# speedup vs baseline: 1.0422x; 1.0422x over previous
import functools

import jax
import jax.numpy as jnp
from jax import lax
from jax.experimental import pallas as pl
from jax.experimental.pallas import tpu as pltpu
from jax.experimental.pallas import tpu_sc as plsc

NUM_SPECIES = 100
EMBED_DIM = 16
N_NODES = 100000

NC = 2
NS = 16
NW = NC * NS

B_MAIN = 3200
B_TAIL = N_NODES - 31 * B_MAIN


@functools.cache
def _make_gather():
    mesh = plsc.VectorSubcoreMesh(
        core_axis_name="c", subcore_axis_name="s", num_cores=NC, num_subcores=NS
    )

    @functools.partial(
        pl.kernel,
        out_type=jax.ShapeDtypeStruct((EMBED_DIM, N_NODES), jnp.float32),
        mesh=mesh,
        scratch_types=[
            pltpu.VMEM((NUM_SPECIES * EMBED_DIM,), jnp.float32),
            pltpu.VMEM((B_MAIN,), jnp.int32),
            pltpu.VMEM((EMBED_DIM, B_MAIN), jnp.float32),
            pltpu.SemaphoreType.DMA,
        ],
        compiler_params=pltpu.CompilerParams(
            use_tc_tiling_on_sc=True, needs_layout_passes=False, skip_device_barrier=True
        ),
    )
    def gather_kernel(table_hbm, idx_hbm, out_hbm, table_v, idx_v, col_v, sem):
        wid = lax.axis_index("s") * NC + lax.axis_index("c")
        base = wid * B_MAIN
        pltpu.sync_copy(table_hbm, table_v)

        def load_group(g16):
            idx16 = idx_v[pl.ds(g16, 16)]
            return tuple(
                plsc.load_gather(table_v, [idx16 + (c * NUM_SPECIES)])
                for c in range(EMBED_DIM)
            )

        def store_group(g16, vs):
            for c in range(EMBED_DIM):
                col_v[c, pl.ds(g16, 16)] = vs[c]

        def do_rows(row0, count):
            def group_body(g, carry):
                prev_g16, prev = carry
                g16 = row0 + g * 16
                new = load_group(g16)
                store_group(prev_g16, prev)
                return (g16, new)

            carry = (row0, load_group(row0))
            carry = lax.fori_loop(1, count // 16, group_body, carry)
            store_group(*carry)

        def do_chunk(parts):
            n_rows = sum(parts)
            pltpu.sync_copy(
                idx_hbm.at[pl.ds(base, n_rows)], idx_v.at[pl.ds(0, n_rows)]
            )
            copies = []
            row0 = 0
            for part in parts:
                do_rows(row0, part)
                n_dma = ((part + 127) // 128) * 128
                copies.append(
                    pltpu.async_copy(
                        col_v.at[:, pl.ds(row0, n_dma)],
                        out_hbm.at[:, pl.ds(base + row0, n_dma)],
                        sem,
                    )
                )
                row0 += part
            for cp in copies:
                cp.wait()

        @pl.when(wid < NW - 1)
        def _():
            do_chunk((896, 768, 768, 768))

        @pl.when(wid == NW - 1)
        def _():
            do_chunk((B_TAIL,))

    return gather_kernel


@jax.jit
def kernel(species_index, embedding_table):
    table_t = embedding_table.T.reshape(NUM_SPECIES * EMBED_DIM)
    out_t = _make_gather()(table_t, species_index.astype(jnp.int32))
    return out_t.T


# speedup vs baseline: 1.0985x; 1.0540x over previous
import functools

import jax
import jax.numpy as jnp
from jax import lax
from jax.experimental import pallas as pl
from jax.experimental.pallas import tpu as pltpu
from jax.experimental.pallas import tpu_sc as plsc

NUM_SPECIES = 100
EMBED_DIM = 16
N_NODES = 100000

NC = 2
NS = 16
NW = NC * NS

B_MAIN = 3200
B_TAIL = N_NODES - 31 * B_MAIN


@functools.cache
def _make_gather():
    mesh = plsc.VectorSubcoreMesh(
        core_axis_name="c", subcore_axis_name="s", num_cores=NC, num_subcores=NS
    )

    @functools.partial(
        pl.kernel,
        out_type=jax.ShapeDtypeStruct((EMBED_DIM, N_NODES), jnp.float32),
        mesh=mesh,
        scratch_types=[
            pltpu.VMEM((NUM_SPECIES * EMBED_DIM,), jnp.float32),
            pltpu.VMEM((B_MAIN,), jnp.int32),
            pltpu.VMEM((EMBED_DIM, B_MAIN), jnp.float32),
            pltpu.SemaphoreType.DMA,
        ],
        compiler_params=pltpu.CompilerParams(
            use_tc_tiling_on_sc=True, needs_layout_passes=False
        ),
    )
    def gather_kernel(table_hbm, idx_hbm, out_hbm, table_v, idx_v, col_v, sem):
        wid = lax.axis_index("s") * NC + lax.axis_index("c")
        base = wid * B_MAIN
        pltpu.sync_copy(table_hbm, table_v)

        def load_group(g16):
            idx16 = idx_v[pl.ds(g16, 16)]
            return tuple(
                plsc.load_gather(table_v, [idx16 + (c * NUM_SPECIES)])
                for c in range(EMBED_DIM)
            )

        def store_group(g16, vs):
            for c in range(EMBED_DIM):
                col_v[c, pl.ds(g16, 16)] = vs[c]

        def do_rows(row0, count):
            def group_body(g, carry):
                prev_g16, prev = carry
                g16 = row0 + g * 16
                idx16 = idx_v[pl.ds(g16, 16)]
                new = []
                for c in range(EMBED_DIM):
                    new.append(
                        plsc.load_gather(table_v, [idx16 + (c * NUM_SPECIES)])
                    )
                    col_v[c, pl.ds(prev_g16, 16)] = prev[c]
                return (g16, tuple(new))

            carry = (row0, load_group(row0))
            carry = lax.fori_loop(1, count // 16, group_body, carry)
            store_group(*carry)

        def do_chunk(parts):
            n_rows = sum(parts)
            pltpu.sync_copy(
                idx_hbm.at[pl.ds(base, n_rows)], idx_v.at[pl.ds(0, n_rows)]
            )
            copies = []
            row0 = 0
            for part in parts:
                do_rows(row0, part)
                n_dma = ((part + 127) // 128) * 128
                copies.append(
                    pltpu.async_copy(
                        col_v.at[:, pl.ds(row0, n_dma)],
                        out_hbm.at[:, pl.ds(base + row0, n_dma)],
                        sem,
                    )
                )
                row0 += part
            for cp in copies:
                cp.wait()

        @pl.when(wid < NW - 1)
        def _():
            do_chunk((896, 768, 768, 768))

        @pl.when(wid == NW - 1)
        def _():
            do_chunk((B_TAIL,))

    return gather_kernel


@jax.jit
def kernel(species_index, embedding_table):
    table_t = embedding_table.T.reshape(NUM_SPECIES * EMBED_DIM)
    out_t = _make_gather()(table_t, species_index.astype(jnp.int32))
    return out_t.T
